# baseline (speedup 1.0000x reference)
import functools

import jax
import jax.numpy as jnp
from jax import lax
from jax.experimental import pallas as pl
from jax.experimental.pallas import tpu as pltpu
from jax.experimental.pallas import tpu_sc as plsc

EMB = 32
NB = 4096
NT = 200
VOCAB = 1000000
B_TOTAL = NB * NT
NUM_WORKERS = 32
BBLK = NB // NUM_WORKERS
PER_WORKER = BBLK * NT


@functools.partial(
    pl.kernel,
    out_type=jax.ShapeDtypeStruct((NT, 4, NUM_WORKERS, 8, 128), jnp.float32),
    mesh=plsc.VectorSubcoreMesh(core_axis_name="c", subcore_axis_name="s"),
    scratch_types=[
        pltpu.VMEM((PER_WORKER,), jnp.int32),
        pltpu.VMEM((BBLK,), jnp.int32),
        pltpu.VMEM((BBLK,), jnp.int32),
        pltpu.VMEM((BBLK, EMB), jnp.float32),
        pltpu.VMEM((BBLK, EMB), jnp.float32),
        pltpu.VMEM((EMB, BBLK), jnp.float32),
        pltpu.VMEM((EMB, BBLK), jnp.float32),
        pltpu.SemaphoreType.DMA,
        pltpu.SemaphoreType.DMA,
        pltpu.SemaphoreType.DMA,
        pltpu.SemaphoreType.DMA,
    ],
    compiler_params=pltpu.CompilerParams(
        use_tc_tiling_on_sc=False, needs_layout_passes=False),
)
def _gather_planar(xf_hbm, trm_hbm, o5_hbm,
                   idx_v, tiA, tiB, rwA, rwB, plA, plB,
                   gsA, gsB, osA, osB):
    wid = lax.axis_index("s") * 2 + lax.axis_index("c")
    base = wid * PER_WORKER

    iota = lax.iota(jnp.int32, 16)
    tvecs = [(b0 + iota) * NT for b0 in range(0, BBLK, 16)]
    rvecs = [b0 + iota for b0 in range(0, BBLK, 16)]
    cvecs = [jnp.full((16,), c, jnp.int32) for c in range(EMB)]

    pltpu.sync_copy(xf_hbm.at[pl.ds(base, PER_WORKER)], idx_v)

    def build_tidx(t, ti):
        for k in range(BBLK // 16):
            g = plsc.load_gather(idx_v, [tvecs[k] + t])
            ti[pl.ds(k * 16, 16)] = g

    def start_gather(ti, rw, sem):
        return pltpu.async_copy(trm_hbm.at[ti], rw, sem)

    def transpose(rw, plv):
        for c in range(EMB):
            for k in range(BBLK // 16):
                g = plsc.load_gather(rw, [rvecs[k], cvecs[c]])
                plv[c, pl.ds(k * 16, 16)] = g

    def start_out(t, plv, sem):
        for a in range(4):
            pltpu.async_copy(
                plv.at[pl.ds(a * 8, 8)], o5_hbm.at[t, a, wid], sem)

    def drain_out(plv, sem):
        for a in range(4):
            pltpu.make_async_copy(
                plv.at[pl.ds(a * 8, 8)], o5_hbm.at[0, a, wid], sem).wait()

    build_tidx(0, tiA)
    gdA = start_gather(tiA, rwA, gsA)

    def body(k, carry):
        t0 = 2 * k
        t1 = t0 + 1
        build_tidx(t1, tiB)
        pltpu.make_async_copy(trm_hbm.at[tiA], rwA, gsA).wait()
        gdB = start_gather(tiB, rwB, gsB)
        del gdB

        @pl.when(k > 0)
        def _():
            drain_out(plA, osA)
        transpose(rwA, plA)
        start_out(t0, plA, osA)

        @pl.when(k < NT // 2 - 1)
        def _():
            build_tidx(t0 + 2, tiA)
        pltpu.make_async_copy(trm_hbm.at[tiB], rwB, gsB).wait()

        @pl.when(k < NT // 2 - 1)
        def _():
            d = start_gather(tiA, rwA, gsA)
            del d

        @pl.when(k > 0)
        def _():
            drain_out(plB, osB)
        transpose(rwB, plB)
        start_out(t1, plB, osB)
        return carry

    lax.fori_loop(0, NT // 2, body, 0)
    drain_out(plA, osA)
    drain_out(plB, osB)


def kernel(X, table):
    xf = X.reshape(-1)
    t128 = table.reshape(VOCAB // 4, 128)
    trm = t128.reshape(VOCAB, EMB)
    o5 = _gather_planar(xf, trm)
    return jnp.transpose(o5, (2, 4, 0, 1, 3)).reshape(NB, NT, EMB)


# speedup vs baseline: 1.2552x; 1.2552x over previous
import functools

import jax
import jax.numpy as jnp
from jax import lax
from jax.experimental import pallas as pl
from jax.experimental.pallas import tpu as pltpu
from jax.experimental.pallas import tpu_sc as plsc

EMB = 32
NB = 4096
NT = 200
VOCAB = 1000000
B_TOTAL = NB * NT
NUM_WORKERS = 32
BBLK = NB // NUM_WORKERS
PER_WORKER = BBLK * NT


@functools.partial(
    pl.kernel,
    out_type=jax.ShapeDtypeStruct((NT, 4, NUM_WORKERS, 8, 128), jnp.float32),
    mesh=plsc.VectorSubcoreMesh(core_axis_name="c", subcore_axis_name="s"),
    scratch_types=[
        pltpu.VMEM((PER_WORKER,), jnp.int32),
        pltpu.VMEM((BBLK,), jnp.int32),
        pltpu.VMEM((BBLK,), jnp.int32),
        pltpu.VMEM((BBLK, EMB), jnp.float32),
        pltpu.VMEM((BBLK, EMB), jnp.float32),
        pltpu.VMEM((EMB, BBLK), jnp.float32),
        pltpu.VMEM((EMB, BBLK), jnp.float32),
        pltpu.SemaphoreType.DMA,
        pltpu.SemaphoreType.DMA,
        pltpu.SemaphoreType.DMA,
        pltpu.SemaphoreType.DMA,
    ],
    compiler_params=pltpu.CompilerParams(
        use_tc_tiling_on_sc=False, needs_layout_passes=False),
)
def _gather_planar(xf_hbm, trm_hbm, o5_hbm,
                   idx_v, tiA, tiB, rwA, rwB, plA, plB,
                   gsA, gsB, osA, osB):
    wid = lax.axis_index("s") * 2 + lax.axis_index("c")
    base = wid * PER_WORKER

    iota = lax.iota(jnp.int32, 16)
    iota_hi = iota + 16

    pltpu.sync_copy(xf_hbm.at[pl.ds(base, PER_WORKER)], idx_v)

    def build_tidx(t, ti):
        @plsc.parallel_loop(0, BBLK // 16, 1, unroll=4)
        def _(k):
            g = plsc.load_gather(idx_v, [(k * 16 + iota) * NT + t])
            ti[pl.ds(k * 16, 16)] = g

    def start_gather(ti, rw, sem):
        return pltpu.async_copy(trm_hbm.at[ti], rw, sem)

    def transpose(rw, plv):
        @plsc.parallel_loop(0, BBLK, 1, unroll=8)
        def _(j):
            jv = jnp.broadcast_to(j, (16,))
            r0 = rw[j, pl.ds(0, 16)]
            r1 = rw[j, pl.ds(16, 16)]
            plsc.store_scatter(plv, [iota, jv], r0)
            plsc.store_scatter(plv, [iota_hi, jv], r1)

    def start_out(t, plv, sem):
        for a in range(4):
            pltpu.async_copy(
                plv.at[pl.ds(a * 8, 8)], o5_hbm.at[t, a, wid], sem)

    def drain_out(plv, sem):
        for a in range(4):
            pltpu.make_async_copy(
                plv.at[pl.ds(a * 8, 8)], o5_hbm.at[0, a, wid], sem).wait()

    build_tidx(0, tiA)
    gdA = start_gather(tiA, rwA, gsA)

    def body(k, carry):
        t0 = 2 * k
        t1 = t0 + 1
        build_tidx(t1, tiB)
        pltpu.make_async_copy(trm_hbm.at[tiA], rwA, gsA).wait()
        gdB = start_gather(tiB, rwB, gsB)
        del gdB

        @pl.when(k > 0)
        def _():
            drain_out(plA, osA)
        transpose(rwA, plA)
        start_out(t0, plA, osA)

        @pl.when(k < NT // 2 - 1)
        def _():
            build_tidx(t0 + 2, tiA)
        pltpu.make_async_copy(trm_hbm.at[tiB], rwB, gsB).wait()

        @pl.when(k < NT // 2 - 1)
        def _():
            d = start_gather(tiA, rwA, gsA)
            del d

        @pl.when(k > 0)
        def _():
            drain_out(plB, osB)
        transpose(rwB, plB)
        start_out(t1, plB, osB)
        return carry

    lax.fori_loop(0, NT // 2, body, 0)
    drain_out(plA, osA)
    drain_out(plB, osB)


def kernel(X, table):
    xf = X.reshape(-1)
    t128 = table.reshape(VOCAB // 4, 128)
    trm = t128.reshape(VOCAB, EMB)
    o5 = _gather_planar(xf, trm)
    return jnp.transpose(o5, (2, 4, 0, 1, 3)).reshape(NB, NT, EMB)


# speedup vs baseline: 1.3128x; 1.0459x over previous
import functools

import jax
import jax.numpy as jnp
from jax import lax
from jax.experimental import pallas as pl
from jax.experimental.pallas import tpu as pltpu
from jax.experimental.pallas import tpu_sc as plsc

EMB = 32
NB = 4096
NT = 200
VOCAB = 1000000
B_TOTAL = NB * NT
NUM_WORKERS = 32
BBLK = NB // NUM_WORKERS
PER_WORKER = BBLK * NT


N_FULL_WIN = VOCAB // 128
TAIL_V0 = N_FULL_WIN * 128


@functools.partial(
    pl.kernel,
    out_type=jax.ShapeDtypeStruct((VOCAB // 4, 128), jnp.float32),
    mesh=plsc.VectorSubcoreMesh(core_axis_name="c", subcore_axis_name="s"),
    scratch_types=[
        pltpu.VMEM((EMB, 128), jnp.float32),
        pltpu.VMEM((EMB, 128), jnp.float32),
        pltpu.VMEM((EMB, 128), jnp.float32),
        pltpu.VMEM((EMB, 128), jnp.float32),
        pltpu.SemaphoreType.DMA,
        pltpu.SemaphoreType.DMA,
        pltpu.SemaphoreType.DMA,
        pltpu.SemaphoreType.DMA,
    ],
    compiler_params=pltpu.CompilerParams(needs_layout_passes=False),
)
def _transpose_table(tt_hbm, tail_hbm, t128_hbm,
                     tiA, tiB, toA, toB, giA, giB, goA, goB):
    wid = lax.axis_index("s") * 2 + lax.axis_index("c")

    iota = lax.iota(jnp.int32, 16)
    iota_hi = iota + 16

    def start_in(q, ti, sem):
        pltpu.async_copy(tt_hbm.at[:, pl.ds(q * 128, 128)], ti, sem)

    def wait_in(ti, sem):
        pltpu.make_async_copy(tt_hbm.at[:, pl.ds(0, 128)], ti, sem).wait()

    def transpose(ti, to, nrows):
        @plsc.parallel_loop(0, nrows * 8, 1, unroll=8)
        def _(n):
            r = n >> 3
            g = n & 7
            rows = iota + (n & 1) * 16
            cols = jnp.broadcast_to(4 * r + ((g >> 1) & 3), (16,))
            v = plsc.load_gather(ti, [rows, cols])
            to[r, pl.ds(g * 16, 16)] = v

    def start_out(q, to, sem, nrows):
        pltpu.async_copy(to.at[pl.ds(0, nrows)],
                         t128_hbm.at[pl.ds(q * 32, nrows)], sem)

    def wait_out(to, sem, nrows):
        pltpu.make_async_copy(to.at[pl.ds(0, nrows)],
                              t128_hbm.at[pl.ds(0, nrows)], sem).wait()

    nwin = jnp.where(wid < N_FULL_WIN - 32 * (N_FULL_WIN // 32),
                     N_FULL_WIN // 32 + 1, N_FULL_WIN // 32)

    start_in(wid, tiA, giA)

    def body(k, carry):
        q0 = wid + 32 * (2 * k)
        q1 = wid + 32 * (2 * k + 1)

        @pl.when(2 * k + 1 < nwin)
        def _():
            start_in(q1, tiB, giB)

        @pl.when(2 * k < nwin)
        def _():
            wait_in(tiA, giA)

            @pl.when(k > 0)
            def _():
                wait_out(toA, goA, EMB)
            transpose(tiA, toA, EMB)
            start_out(q0, toA, goA, EMB)

        @pl.when(2 * k + 2 < nwin)
        def _():
            start_in(wid + 32 * (2 * k + 2), tiA, giA)

        @pl.when(2 * k + 1 < nwin)
        def _():
            wait_in(tiB, giB)

            @pl.when(k > 0)
            def _():
                wait_out(toB, goB, EMB)
            transpose(tiB, toB, EMB)
            start_out(q1, toB, goB, EMB)
        return carry

    lax.fori_loop(0, (N_FULL_WIN // 32 + 2) // 2, body, 0)
    wait_out(toA, goA, EMB)
    wait_out(toB, goB, EMB)

    @pl.when(wid == 4)
    def _():
        pltpu.sync_copy(tail_hbm, toA.at[pl.ds(0, 16)])
        pltpu.sync_copy(toA.at[pl.ds(0, 16)],
                        t128_hbm.at[pl.ds(N_FULL_WIN * 32, 16)])


@functools.partial(
    pl.kernel,
    out_type=jax.ShapeDtypeStruct((NT, 4, NUM_WORKERS, 8, 128), jnp.float32),
    mesh=plsc.VectorSubcoreMesh(core_axis_name="c", subcore_axis_name="s"),
    scratch_types=[
        pltpu.VMEM((PER_WORKER,), jnp.int32),
        pltpu.VMEM((BBLK,), jnp.int32),
        pltpu.VMEM((BBLK,), jnp.int32),
        pltpu.VMEM((BBLK, EMB), jnp.float32),
        pltpu.VMEM((BBLK, EMB), jnp.float32),
        pltpu.VMEM((EMB, BBLK), jnp.float32),
        pltpu.VMEM((EMB, BBLK), jnp.float32),
        pltpu.SemaphoreType.DMA,
        pltpu.SemaphoreType.DMA,
        pltpu.SemaphoreType.DMA,
        pltpu.SemaphoreType.DMA,
    ],
    compiler_params=pltpu.CompilerParams(
        use_tc_tiling_on_sc=False, needs_layout_passes=False),
)
def _gather_planar(xf_hbm, trm_hbm, o5_hbm,
                   idx_v, tiA, tiB, rwA, rwB, plA, plB,
                   gsA, gsB, osA, osB):
    wid = lax.axis_index("s") * 2 + lax.axis_index("c")
    base = wid * PER_WORKER

    iota = lax.iota(jnp.int32, 16)
    iota_hi = iota + 16

    pltpu.sync_copy(xf_hbm.at[pl.ds(base, PER_WORKER)], idx_v)

    def build_tidx(t, ti):
        @plsc.parallel_loop(0, BBLK // 16, 1, unroll=4)
        def _(k):
            g = plsc.load_gather(idx_v, [(k * 16 + iota) * NT + t])
            ti[pl.ds(k * 16, 16)] = g

    def start_gather(ti, rw, sem):
        return pltpu.async_copy(trm_hbm.at[ti], rw, sem)

    def transpose(rw, plv):
        @plsc.parallel_loop(0, BBLK, 1, unroll=8)
        def _(j):
            jv = jnp.broadcast_to(j, (16,))
            r0 = rw[j, pl.ds(0, 16)]
            r1 = rw[j, pl.ds(16, 16)]
            plsc.store_scatter(plv, [iota, jv], r0)
            plsc.store_scatter(plv, [iota_hi, jv], r1)

    def start_out(t, plv, sem):
        for a in range(4):
            pltpu.async_copy(
                plv.at[pl.ds(a * 8, 8)], o5_hbm.at[t, a, wid], sem)

    def drain_out(plv, sem):
        for a in range(4):
            pltpu.make_async_copy(
                plv.at[pl.ds(a * 8, 8)], o5_hbm.at[0, a, wid], sem).wait()

    build_tidx(0, tiA)
    gdA = start_gather(tiA, rwA, gsA)

    def body(k, carry):
        t0 = 2 * k
        t1 = t0 + 1
        build_tidx(t1, tiB)
        pltpu.make_async_copy(trm_hbm.at[tiA], rwA, gsA).wait()
        gdB = start_gather(tiB, rwB, gsB)
        del gdB

        @pl.when(k > 0)
        def _():
            drain_out(plA, osA)
        transpose(rwA, plA)
        start_out(t0, plA, osA)

        @pl.when(k < NT // 2 - 1)
        def _():
            build_tidx(t0 + 2, tiA)
        pltpu.make_async_copy(trm_hbm.at[tiB], rwB, gsB).wait()

        @pl.when(k < NT // 2 - 1)
        def _():
            d = start_gather(tiA, rwA, gsA)
            del d

        @pl.when(k > 0)
        def _():
            drain_out(plB, osB)
        transpose(rwB, plB)
        start_out(t1, plB, osB)
        return carry

    lax.fori_loop(0, NT // 2, body, 0)
    drain_out(plA, osA)
    drain_out(plB, osB)


def kernel(X, table):
    xf = X.reshape(-1)
    tail128 = table[TAIL_V0:].reshape(16, 128)
    t128 = _transpose_table(table.T, tail128)
    trm = t128.reshape(VOCAB, EMB)
    o5 = _gather_planar(xf, trm)
    return jnp.transpose(o5, (2, 4, 0, 1, 3)).reshape(NB, NT, EMB)
